# baseline (device time: 248767 ns/iter reference)
import functools

import jax
import jax.numpy as jnp
from jax import lax
from jax.experimental import pallas as pl
from jax.experimental.pallas import tpu as pltpu

N_DEV = 4
SQ = 2048
SKV = 2048
HQ = 8
HH = HQ // 2
DH = 128
DM = 1024
DMH = DM // 2
BLK = 64
QC = 128
N_CHUNK = SQ // QC
HPC = max(1, HQ // N_CHUNK)
SCALE = 0.08838834764831843


def kernel(x, Wq, K_ext, V_ext, Wo):
    xb = x[0].astype(jnp.bfloat16)
    wq = (Wq * SCALE).astype(jnp.bfloat16)
    woT = Wo.T.astype(jnp.bfloat16)

    def body(x_ref, k_hbm, v_hbm, wq_ref, woT_ref, out_ref,
             comm_cw_ref, comm_ccw_ref, kg_ref, vg_ref, kst_ref, vst_ref,
             send_sems, recv_sems, kv_sems):
        my_pos = lax.axis_index("i")
        left = lax.rem(my_pos + N_DEV - 1, N_DEV)
        right = lax.rem(my_pos + 1, N_DEV)

        barrier = pltpu.get_barrier_semaphore()
        for nbr in (left, right):
            pl.semaphore_signal(barrier, inc=1, device_id=(nbr,),
                                device_id_type=pl.DeviceIdType.MESH)
        pl.semaphore_wait(barrier, 2)

        comm_cw_ref[0, 0] = wq_ref[:, :DMH]
        comm_cw_ref[0, 1] = woT_ref[:, :DMH]
        comm_ccw_ref[0, 0] = wq_ref[:, DMH:]
        comm_ccw_ref[0, 1] = woT_ref[:, DMH:]
        out_ref[...] = jnp.zeros_like(out_ref)

        qb_i = lax.broadcasted_iota(jnp.int32, (QC, QC), 0) // BLK
        kb_i = lax.broadcasted_iota(jnp.int32, (QC, QC), 1) // BLK
        diag_neg = jnp.where(kb_i <= qb_i, 0.0, -1e9).astype(jnp.float32)

        def stage_start(g, hd, slot):
            pltpu.make_async_copy(
                k_hbm.at[my_pos, :, g * HQ + hd, :],
                kst_ref.at[slot], kv_sems.at[0, slot]).start()
            pltpu.make_async_copy(
                v_hbm.at[my_pos, :, g * HQ + hd, :],
                vst_ref.at[slot], kv_sems.at[1, slot]).start()

        def stage_finish(g, gbuf, hd, slot):
            pltpu.make_async_copy(
                k_hbm.at[my_pos, :, g * HQ + hd, :],
                kst_ref.at[slot], kv_sems.at[0, slot]).wait()
            pltpu.make_async_copy(
                v_hbm.at[my_pos, :, g * HQ + hd, :],
                vst_ref.at[slot], kv_sems.at[1, slot]).wait()
            kg_ref[gbuf, hd] = kst_ref[slot].astype(jnp.bfloat16)
            vg_ref[gbuf, hd] = vst_ref[slot].astype(jnp.bfloat16)

        stage_start(my_pos, 0, 0)
        stage_start(my_pos, 1, 1)
        for hd in range(HQ):
            stage_finish(my_pos, 0, hd, hd % 2)
            if hd + 2 < HQ:
                stage_start(my_pos, hd + 2, hd % 2)

        def hop(h, carry):
            gslot = lax.rem(h, 2)
            nslot = 1 - gslot
            g_cw = lax.rem(my_pos - h + N_DEV, N_DEV)
            g_ccw = lax.rem(my_pos + h, N_DEV)
            g_cw_n = lax.rem(my_pos - h - 1 + 2 * N_DEV, N_DEV)
            g_ccw_n = lax.rem(my_pos + h + 1, N_DEV)
            nxt = jnp.minimum(h + 1, N_DEV - 1)

            rdma_cw = pltpu.make_async_remote_copy(
                src_ref=comm_cw_ref.at[h],
                dst_ref=comm_cw_ref.at[nxt],
                send_sem=send_sems.at[0, h],
                recv_sem=recv_sems.at[0, nxt],
                device_id=(right,),
                device_id_type=pl.DeviceIdType.MESH,
            )
            rdma_ccw = pltpu.make_async_remote_copy(
                src_ref=comm_ccw_ref.at[h],
                dst_ref=comm_ccw_ref.at[nxt],
                send_sem=send_sems.at[1, h],
                recv_sem=recv_sems.at[1, nxt],
                device_id=(left,),
                device_id_type=pl.DeviceIdType.MESH,
            )

            def src_g(j):
                return g_cw_n if j < HH else g_ccw_n

            @pl.when(h < N_DEV - 1)
            def _():
                rdma_cw.start()
                rdma_ccw.start()
                stage_start(src_g(0), 0, 0)
                stage_start(src_g(1), 1, 1)

            wq_cw = comm_cw_ref[h, 0]
            woT_cw = comm_cw_ref[h, 1]
            wq_ccw = comm_ccw_ref[h, 0]
            woT_ccw = comm_ccw_ref[h, 1]

            cc = (((1,), (1,)), ((), ()))
            for c in range(N_CHUNK):
                xc = x_ref[c * QC:(c + 1) * QC, :]
                q_cw = jnp.dot(
                    xc, wq_cw,
                    preferred_element_type=jnp.float32).astype(jnp.bfloat16)
                q_ccw = jnp.dot(
                    xc, wq_ccw,
                    preferred_element_type=jnp.float32).astype(jnp.bfloat16)
                ctx_parts = []
                for hd in range(HQ):
                    qh = q_cw if hd < HH else q_ccw
                    j = hd % HH
                    q_hd = qh[:, j * DH:(j + 1) * DH]
                    p_d = jnp.exp(lax.dot_general(
                        q_hd, kg_ref[gslot, hd, c * QC:(c + 1) * QC], cc,
                        preferred_element_type=jnp.float32) + diag_neg)
                    denom = jnp.sum(p_d, axis=1, keepdims=True)
                    ctx = jnp.dot(
                        p_d.astype(jnp.bfloat16),
                        vg_ref[gslot, hd, c * QC:(c + 1) * QC],
                        preferred_element_type=jnp.float32)
                    if c > 0:
                        p_v = jnp.exp(lax.dot_general(
                            q_hd, kg_ref[gslot, hd, :c * QC], cc,
                            preferred_element_type=jnp.float32))
                        denom = denom + jnp.sum(p_v, axis=1, keepdims=True)
                        ctx = ctx + jnp.dot(
                            p_v.astype(jnp.bfloat16),
                            vg_ref[gslot, hd, :c * QC],
                            preferred_element_type=jnp.float32)
                    ctx_parts.append((ctx / denom).astype(jnp.bfloat16))
                ctx_cw = jnp.concatenate(ctx_parts[:HH], axis=1)
                ctx_ccw = jnp.concatenate(ctx_parts[HH:], axis=1)
                out_ref[c * QC:(c + 1) * QC, :] += (
                    lax.dot_general(ctx_cw, woT_cw, cc,
                                    preferred_element_type=jnp.float32)
                    + lax.dot_general(ctx_ccw, woT_ccw, cc,
                                      preferred_element_type=jnp.float32))

                for j in range(c * HPC, min((c + 1) * HPC, HQ)):
                    @pl.when(h < N_DEV - 1)
                    def _(j=j):
                        stage_finish(src_g(j), nslot, j, j % 2)
                        if j + 2 < HQ:
                            stage_start(src_g(j + 2), j + 2, j % 2)

            @pl.when(h < N_DEV - 1)
            def _():
                rdma_cw.wait()
                rdma_ccw.wait()

            return carry

        lax.fori_loop(0, N_DEV, hop, 0)

        @functools.partial(pl.run_scoped,
                           sem2=pltpu.SemaphoreType.REGULAR)
        def _(sem2):
            for nbr in (left, right):
                pl.semaphore_signal(sem2, inc=1, device_id=(nbr,),
                                    device_id_type=pl.DeviceIdType.MESH)
            pl.semaphore_wait(sem2, 2)

    out = pl.pallas_call(
        body,
        out_shape=jax.ShapeDtypeStruct((SQ, DM), jnp.float32),
        in_specs=[
            pl.BlockSpec(memory_space=pltpu.VMEM),
            pl.BlockSpec(memory_space=pl.ANY),
            pl.BlockSpec(memory_space=pl.ANY),
            pl.BlockSpec(memory_space=pltpu.VMEM),
            pl.BlockSpec(memory_space=pltpu.VMEM),
        ],
        out_specs=pl.BlockSpec(memory_space=pltpu.VMEM),
        scratch_shapes=[
            pltpu.VMEM((N_DEV, 2, DM, DMH), jnp.bfloat16),
            pltpu.VMEM((N_DEV, 2, DM, DMH), jnp.bfloat16),
            pltpu.VMEM((2, HQ, SKV, DH), jnp.bfloat16),
            pltpu.VMEM((2, HQ, SKV, DH), jnp.bfloat16),
            pltpu.VMEM((2, SKV, DH), jnp.float32),
            pltpu.VMEM((2, SKV, DH), jnp.float32),
            pltpu.SemaphoreType.DMA((2, N_DEV)),
            pltpu.SemaphoreType.DMA((2, N_DEV)),
            pltpu.SemaphoreType.DMA((2, 2)),
        ],
        compiler_params=pltpu.CompilerParams(
            collective_id=0,
            vmem_limit_bytes=63 * 1024 * 1024,
        ),
    )(xb, K_ext, V_ext, wq, woT)
    return out[None]


# device time: 173459 ns/iter; 1.4342x vs baseline; 1.4342x over previous
import functools

import jax
import jax.numpy as jnp
from jax import lax
from jax.experimental import pallas as pl
from jax.experimental.pallas import tpu as pltpu

N_DEV = 4
SQ = 2048
SKV = 2048
HQ = 8
HH = HQ // 2
DH = 128
DM = 1024
DMH = DM // 2
BLK = 64
QC = 256
N_CHUNK = SQ // QC
HPC = max(1, HQ // N_CHUNK)
SCALE = 0.08838834764831843


def kernel(x, Wq, K_ext, V_ext, Wo):
    xb = x[0].astype(jnp.bfloat16)
    wq = (Wq * SCALE).astype(jnp.bfloat16)
    woT = Wo.T.astype(jnp.bfloat16)

    def body(x_ref, k_hbm, v_hbm, wq_ref, woT_ref, out_ref,
             comm_cw_ref, comm_ccw_ref, kg_ref, vg_ref, kst_ref, vst_ref,
             send_sems, recv_sems, kv_sems):
        my_pos = lax.axis_index("i")
        left = lax.rem(my_pos + N_DEV - 1, N_DEV)
        right = lax.rem(my_pos + 1, N_DEV)

        barrier = pltpu.get_barrier_semaphore()
        for nbr in (left, right):
            pl.semaphore_signal(barrier, inc=1, device_id=(nbr,),
                                device_id_type=pl.DeviceIdType.MESH)
        pl.semaphore_wait(barrier, 2)

        comm_cw_ref[0, 0] = wq_ref[:, :DMH]
        comm_cw_ref[0, 1] = woT_ref[:, :DMH]
        comm_ccw_ref[0, 0] = wq_ref[:, DMH:]
        comm_ccw_ref[0, 1] = woT_ref[:, DMH:]
        out_ref[...] = jnp.zeros_like(out_ref)

        qb_i = lax.broadcasted_iota(jnp.int32, (QC, QC), 0) // BLK
        kb_i = lax.broadcasted_iota(jnp.int32, (QC, QC), 1) // BLK
        diag_neg = jnp.where(kb_i <= qb_i, 0.0, -1e9).astype(jnp.float32)

        def stage_start(g, hd, slot):
            pltpu.make_async_copy(
                k_hbm.at[my_pos, :, g * HQ + hd, :],
                kst_ref.at[slot], kv_sems.at[0, slot]).start()
            pltpu.make_async_copy(
                v_hbm.at[my_pos, :, g * HQ + hd, :],
                vst_ref.at[slot], kv_sems.at[1, slot]).start()

        def stage_finish(g, gbuf, hd, slot):
            pltpu.make_async_copy(
                k_hbm.at[my_pos, :, g * HQ + hd, :],
                kst_ref.at[slot], kv_sems.at[0, slot]).wait()
            pltpu.make_async_copy(
                v_hbm.at[my_pos, :, g * HQ + hd, :],
                vst_ref.at[slot], kv_sems.at[1, slot]).wait()
            kg_ref[gbuf, hd] = kst_ref[slot].astype(jnp.bfloat16)
            vg_ref[gbuf, hd] = vst_ref[slot].astype(jnp.bfloat16)

        stage_start(my_pos, 0, 0)
        stage_start(my_pos, 1, 1)
        for hd in range(HQ):
            stage_finish(my_pos, 0, hd, hd % 2)
            if hd + 2 < HQ:
                stage_start(my_pos, hd + 2, hd % 2)

        def hop(h, carry):
            gslot = lax.rem(h, 2)
            nslot = 1 - gslot
            g_cw = lax.rem(my_pos - h + N_DEV, N_DEV)
            g_ccw = lax.rem(my_pos + h, N_DEV)
            g_cw_n = lax.rem(my_pos - h - 1 + 2 * N_DEV, N_DEV)
            g_ccw_n = lax.rem(my_pos + h + 1, N_DEV)
            nxt = jnp.minimum(h + 1, N_DEV - 1)

            rdma_cw = pltpu.make_async_remote_copy(
                src_ref=comm_cw_ref.at[h],
                dst_ref=comm_cw_ref.at[nxt],
                send_sem=send_sems.at[0, h],
                recv_sem=recv_sems.at[0, nxt],
                device_id=(right,),
                device_id_type=pl.DeviceIdType.MESH,
            )
            rdma_ccw = pltpu.make_async_remote_copy(
                src_ref=comm_ccw_ref.at[h],
                dst_ref=comm_ccw_ref.at[nxt],
                send_sem=send_sems.at[1, h],
                recv_sem=recv_sems.at[1, nxt],
                device_id=(left,),
                device_id_type=pl.DeviceIdType.MESH,
            )

            def src_g(j):
                return g_cw_n if j < HH else g_ccw_n

            @pl.when(h < N_DEV - 1)
            def _():
                rdma_cw.start()
                rdma_ccw.start()
                stage_start(src_g(0), 0, 0)
                stage_start(src_g(1), 1, 1)

            wq_cw = comm_cw_ref[h, 0]
            woT_cw = comm_cw_ref[h, 1]
            wq_ccw = comm_ccw_ref[h, 0]
            woT_ccw = comm_ccw_ref[h, 1]

            cc = (((1,), (1,)), ((), ()))
            for c in range(N_CHUNK):
                xc = x_ref[c * QC:(c + 1) * QC, :]
                q_cw = jnp.dot(
                    xc, wq_cw,
                    preferred_element_type=jnp.float32).astype(jnp.bfloat16)
                q_ccw = jnp.dot(
                    xc, wq_ccw,
                    preferred_element_type=jnp.float32).astype(jnp.bfloat16)
                ctx_parts = []
                for hd in range(HQ):
                    qh = q_cw if hd < HH else q_ccw
                    j = hd % HH
                    q_hd = qh[:, j * DH:(j + 1) * DH]
                    p_d = jnp.exp(lax.dot_general(
                        q_hd, kg_ref[gslot, hd, c * QC:(c + 1) * QC], cc,
                        preferred_element_type=jnp.float32) + diag_neg)
                    denom = jnp.sum(p_d, axis=1, keepdims=True)
                    ctx = jnp.dot(
                        p_d.astype(jnp.bfloat16),
                        vg_ref[gslot, hd, c * QC:(c + 1) * QC],
                        preferred_element_type=jnp.float32)
                    if c > 0:
                        p_v = jnp.exp(lax.dot_general(
                            q_hd, kg_ref[gslot, hd, :c * QC], cc,
                            preferred_element_type=jnp.float32))
                        denom = denom + jnp.sum(p_v, axis=1, keepdims=True)
                        ctx = ctx + jnp.dot(
                            p_v.astype(jnp.bfloat16),
                            vg_ref[gslot, hd, :c * QC],
                            preferred_element_type=jnp.float32)
                    ctx_parts.append((ctx / denom).astype(jnp.bfloat16))
                ctx_cw = jnp.concatenate(ctx_parts[:HH], axis=1)
                ctx_ccw = jnp.concatenate(ctx_parts[HH:], axis=1)
                out_ref[c * QC:(c + 1) * QC, :] += (
                    lax.dot_general(ctx_cw, woT_cw, cc,
                                    preferred_element_type=jnp.float32)
                    + lax.dot_general(ctx_ccw, woT_ccw, cc,
                                      preferred_element_type=jnp.float32))

                for j in range(c * HPC, min((c + 1) * HPC, HQ)):
                    @pl.when(h < N_DEV - 1)
                    def _(j=j):
                        stage_finish(src_g(j), nslot, j, j % 2)
                        if j + 2 < HQ:
                            stage_start(src_g(j + 2), j + 2, j % 2)

            @pl.when(h < N_DEV - 1)
            def _():
                rdma_cw.wait()
                rdma_ccw.wait()

            return carry

        lax.fori_loop(0, N_DEV, hop, 0)

        @functools.partial(pl.run_scoped,
                           sem2=pltpu.SemaphoreType.REGULAR)
        def _(sem2):
            for nbr in (left, right):
                pl.semaphore_signal(sem2, inc=1, device_id=(nbr,),
                                    device_id_type=pl.DeviceIdType.MESH)
            pl.semaphore_wait(sem2, 2)

    out = pl.pallas_call(
        body,
        out_shape=jax.ShapeDtypeStruct((SQ, DM), jnp.float32),
        in_specs=[
            pl.BlockSpec(memory_space=pltpu.VMEM),
            pl.BlockSpec(memory_space=pl.ANY),
            pl.BlockSpec(memory_space=pl.ANY),
            pl.BlockSpec(memory_space=pltpu.VMEM),
            pl.BlockSpec(memory_space=pltpu.VMEM),
        ],
        out_specs=pl.BlockSpec(memory_space=pltpu.VMEM),
        scratch_shapes=[
            pltpu.VMEM((N_DEV, 2, DM, DMH), jnp.bfloat16),
            pltpu.VMEM((N_DEV, 2, DM, DMH), jnp.bfloat16),
            pltpu.VMEM((2, HQ, SKV, DH), jnp.bfloat16),
            pltpu.VMEM((2, HQ, SKV, DH), jnp.bfloat16),
            pltpu.VMEM((2, SKV, DH), jnp.float32),
            pltpu.VMEM((2, SKV, DH), jnp.float32),
            pltpu.SemaphoreType.DMA((2, N_DEV)),
            pltpu.SemaphoreType.DMA((2, N_DEV)),
            pltpu.SemaphoreType.DMA((2, 2)),
        ],
        compiler_params=pltpu.CompilerParams(
            collective_id=0,
            vmem_limit_bytes=63 * 1024 * 1024,
        ),
    )(xb, K_ext, V_ext, wq, woT)
    return out[None]
